# Initial kernel scaffold; baseline (speedup 1.0000x reference)
import jax
import jax.numpy as jnp
from jax.experimental import pallas as pl


def kernel(feat, edge_index, W_neigh, W_self, b_self):
    raise NotImplementedError("write your pallas kernel here")


# speedup vs baseline: 5.6004x; 5.6004x over previous
import functools

import jax
import jax.numpy as jnp
from jax import lax
from jax.experimental import pallas as pl
from jax.experimental.pallas import tpu as pltpu
from jax.experimental.pallas import tpu_sc as plsc

N_NODES = 10000
N_EDGES = 320000
D = 128

NC = 2
NS = 16
NW = NC * NS
CH = 128
NCH = 79
E_PAD = NW * NCH * CH
ROWS = 10240
RPT = ROWS // NS
BLK = 1000


def _mm_body(feat_ref, wn_ref, ws_ref, b_ref, hsrc_ref, hself_ref):
    f = feat_ref[...]
    dn = (((1,), (1,)), ((), ()))
    hsrc_ref[...] = lax.dot_general(f, wn_ref[...], dn,
                                    preferred_element_type=jnp.float32)
    hself_ref[...] = lax.dot_general(f, ws_ref[...], dn,
                                     preferred_element_type=jnp.float32) + b_ref[...]


def _agg_body(h_hbm, src_hbm, dst_hbm, acc_out, deg_out,
              src_v, dst_v, rows_v, ones_v, zdeg_v, acc_sh, deg_sh, sem):
    c = lax.axis_index("c")
    s = lax.axis_index("s")
    wid = c * NS + s

    def _zrow(i, _):
        for j in range(8):
            rows_v[i, pl.ds(j * 16, 16)] = jnp.zeros((16,), jnp.float32)
        return 0
    lax.fori_loop(0, CH, _zrow, 0)
    for j in range(8):
        ones_v[pl.ds(j * 16, 16)] = jnp.ones((16,), jnp.float32)
    def _zdeg(i, _):
        zdeg_v[pl.ds(i * 16, 16)] = jnp.zeros((16,), jnp.float32)
        return 0
    lax.fori_loop(0, RPT // 16, _zdeg, 0)

    base = s * RPT
    for k in range(RPT // CH):
        pltpu.sync_copy(rows_v, acc_sh.at[pl.ds(base + k * CH, CH)])
    pltpu.sync_copy(zdeg_v, deg_sh.at[pl.ds(base, RPT)])
    plsc.subcore_barrier()

    pltpu.sync_copy(src_hbm.at[wid], src_v)
    pltpu.sync_copy(dst_hbm.at[wid], dst_v)

    def _step(j, _):
        pltpu.async_copy(h_hbm.at[src_v.at[j]], rows_v, sem).wait()
        pltpu.sync_copy(rows_v, acc_sh.at[dst_v.at[j]], add=True)
        pltpu.sync_copy(ones_v, deg_sh.at[dst_v.at[j]], add=True)
        return 0
    lax.fori_loop(0, NCH, _step, 0)
    plsc.subcore_barrier()

    for k in range(RPT // CH):
        pltpu.sync_copy(acc_sh.at[pl.ds(base + k * CH, CH)],
                        acc_out.at[c].at[pl.ds(base + k * CH, CH)])
    pltpu.sync_copy(deg_sh.at[pl.ds(base, RPT)],
                    deg_out.at[c].at[pl.ds(base, RPT)])


_agg = functools.partial(
    pl.kernel,
    out_type=[
        jax.ShapeDtypeStruct((NC, ROWS, D), jnp.float32),
        jax.ShapeDtypeStruct((NC, ROWS), jnp.float32),
    ],
    mesh=plsc.VectorSubcoreMesh(core_axis_name="c", subcore_axis_name="s"),
    scratch_types=[
        pltpu.VMEM((NCH, CH), jnp.int32),
        pltpu.VMEM((NCH, CH), jnp.int32),
        pltpu.VMEM((CH, D), jnp.float32),
        pltpu.VMEM((CH,), jnp.float32),
        pltpu.VMEM((RPT,), jnp.float32),
        pltpu.VMEM_SHARED((ROWS, D), jnp.float32),
        pltpu.VMEM_SHARED((ROWS,), jnp.float32),
        pltpu.SemaphoreType.DMA,
    ],
)(_agg_body)


def _ep_body(hself_ref, acc_ref, deg_ref, out_ref):
    ssum = acc_ref[0] + acc_ref[1]
    dr = deg_ref[...]
    d = dr[:, 0:1] + dr[:, 1:2]
    out_ref[...] = hself_ref[...] + ssum / jnp.maximum(d, 1.0)


@jax.jit
def kernel(feat, edge_index, W_neigh, W_self, b_self):
    src = edge_index[0].astype(jnp.int32)
    dst = edge_index[1].astype(jnp.int32)
    pad = E_PAD - N_EDGES
    src_p = jnp.concatenate([src, jnp.zeros((pad,), jnp.int32)])
    dst_p = jnp.concatenate([dst, jnp.full((pad,), N_NODES, jnp.int32)])
    src_p = src_p.reshape(NW, NCH, CH)
    dst_p = dst_p.reshape(NW, NCH, CH)

    grid = N_NODES // BLK
    h_src, h_self = pl.pallas_call(
        _mm_body,
        grid=(grid,),
        in_specs=[
            pl.BlockSpec((BLK, D), lambda i: (i, 0)),
            pl.BlockSpec((D, D), lambda i: (0, 0)),
            pl.BlockSpec((D, D), lambda i: (0, 0)),
            pl.BlockSpec((1, D), lambda i: (0, 0)),
        ],
        out_specs=[
            pl.BlockSpec((BLK, D), lambda i: (i, 0)),
            pl.BlockSpec((BLK, D), lambda i: (i, 0)),
        ],
        out_shape=[
            jax.ShapeDtypeStruct((N_NODES, D), jnp.float32),
            jax.ShapeDtypeStruct((N_NODES, D), jnp.float32),
        ],
    )(feat, W_neigh, W_self, b_self.reshape(1, D))

    acc, deg = _agg(h_src, src_p, dst_p)
    deg_t = deg.T

    rst = pl.pallas_call(
        _ep_body,
        grid=(grid,),
        in_specs=[
            pl.BlockSpec((BLK, D), lambda i: (i, 0)),
            pl.BlockSpec((NC, BLK, D), lambda i: (0, i, 0)),
            pl.BlockSpec((BLK, NC), lambda i: (i, 0)),
        ],
        out_specs=pl.BlockSpec((BLK, D), lambda i: (i, 0)),
        out_shape=jax.ShapeDtypeStruct((N_NODES, D), jnp.float32),
    )(h_self, acc, deg_t)
    return rst
